# Initial kernel scaffold; baseline (speedup 1.0000x reference)
import jax
import jax.numpy as jnp
from jax.experimental import pallas as pl


def kernel(Omega, W_pre, H, L, Z):
    raise NotImplementedError("write your pallas kernel here")


# speedup vs baseline: 559.5114x; 559.5114x over previous
import jax
import jax.numpy as jnp
from jax import lax
from jax.experimental import pallas as pl
from jax.experimental.pallas import tpu as pltpu

_ROWS, _RANK, _COLS = 4096, 64, 512
_LTR = 0.01
_ALPHA = 0.1
_TOL = 1e-8
_MAXIT = 20
_K1 = 1.0
_KPOW = 48
_BLK = 512
_NBLK = _ROWS // _BLK


def _setup_kernel(om_ref, z_ref, h_ref, ht_ref, v_ref, lam_ref):
    m = om_ref[...].astype(jnp.float32)
    h = h_ref[...]
    ht = ht_ref[...]
    v_ref[...] = jnp.dot(m * z_ref[...], ht, preferred_element_type=jnp.float32)

    def apply_m(v):
        t = jnp.dot(v, h, preferred_element_type=jnp.float32) * m
        return jnp.dot(t, ht, preferred_element_type=jnp.float32) + _ALPHA * v

    v0 = jnp.dot(m, (h * h).T, preferred_element_type=jnp.float32) + _ALPHA

    def body(_, v):
        w = apply_m(v)
        return w / jnp.max(jnp.abs(w), axis=1, keepdims=True)

    v = lax.fori_loop(0, _KPOW, body, v0)
    w = apply_m(v)
    lam_ref[...] = (jnp.sum(v * w, axis=1, keepdims=True)
                    / jnp.sum(v * v, axis=1, keepdims=True))


def _iter_kernel(done_ref, w_ref, l_ref, om_ref, h_ref, ht_ref, v_ref, lam_ref,
                 wout_ref, sse_ref):
    pid = pl.program_id(0)
    w_all = w_ref[...]
    wb = w_ref[pl.ds(pid * _BLK, _BLK), :]
    g = jnp.dot(l_ref[...], w_all.astype(jnp.bfloat16),
                preferred_element_type=jnp.float32)
    m = om_ref[...].astype(jnp.float32)
    t = jnp.dot(wb, h_ref[...], preferred_element_type=jnp.float32) * m
    u = jnp.dot(t, ht_ref[...], preferred_element_type=jnp.float32)
    lam = lam_ref[...]
    wn = jax.nn.relu(wb + (v_ref[...] - u - _ALPHA * wb - _LTR * g) / lam)
    frozen = done_ref[0] != 0
    wout_ref[...] = jnp.where(frozen, wb, wn)
    d = wn - wb
    sse_ref[...] = jnp.full((1, 1, 128), jnp.sum(d * d), jnp.float32)


def _setup_call(Omega, Z, H, Ht):
    return pl.pallas_call(
        _setup_kernel,
        grid=(_NBLK,),
        in_specs=[
            pl.BlockSpec((_BLK, _COLS), lambda i: (i, 0)),
            pl.BlockSpec((_BLK, _COLS), lambda i: (i, 0)),
            pl.BlockSpec((_RANK, _COLS), lambda i: (0, 0)),
            pl.BlockSpec((_COLS, _RANK), lambda i: (0, 0)),
        ],
        out_specs=[
            pl.BlockSpec((_BLK, _RANK), lambda i: (i, 0)),
            pl.BlockSpec((_BLK, 1), lambda i: (i, 0)),
        ],
        out_shape=[
            jax.ShapeDtypeStruct((_ROWS, _RANK), jnp.float32),
            jax.ShapeDtypeStruct((_ROWS, 1), jnp.float32),
        ],
        compiler_params=pltpu.CompilerParams(
            dimension_semantics=("arbitrary",),
        ),
        name="wlayer_setup",
    )(Omega, Z, H, Ht)


def _iter_call(done, W, L_bf, m_bf, H, Ht, V, lam):
    return pl.pallas_call(
        _iter_kernel,
        grid=(_NBLK,),
        in_specs=[
            pl.BlockSpec(memory_space=pltpu.SMEM),
            pl.BlockSpec((_ROWS, _RANK), lambda i: (0, 0)),
            pl.BlockSpec((_BLK, _ROWS), lambda i: (i, 0)),
            pl.BlockSpec((_BLK, _COLS), lambda i: (i, 0)),
            pl.BlockSpec((_RANK, _COLS), lambda i: (0, 0)),
            pl.BlockSpec((_COLS, _RANK), lambda i: (0, 0)),
            pl.BlockSpec((_BLK, _RANK), lambda i: (i, 0)),
            pl.BlockSpec((_BLK, 1), lambda i: (i, 0)),
        ],
        out_specs=[
            pl.BlockSpec((_BLK, _RANK), lambda i: (i, 0)),
            pl.BlockSpec((1, 1, 128), lambda i: (i, 0, 0)),
        ],
        out_shape=[
            jax.ShapeDtypeStruct((_ROWS, _RANK), jnp.float32),
            jax.ShapeDtypeStruct((_NBLK, 1, 128), jnp.float32),
        ],
        compiler_params=pltpu.CompilerParams(
            dimension_semantics=("arbitrary",),
        ),
        name="wlayer_iter",
    )(done, W, L_bf, m_bf, H, Ht, V, lam)


def kernel(Omega, W_pre, H, L, Z):
    L_bf = L.astype(jnp.bfloat16)
    m_bf = Omega.astype(jnp.bfloat16)
    Ht = H.T
    denom = jnp.asarray(_RANK * _ROWS, jnp.float32)

    V, lam = _setup_call(Omega, Z, H, Ht)

    def step(carry, _):
        W, done = carry
        wn, sse = _iter_call(done.astype(jnp.int32).reshape(1), W, L_bf, m_bf,
                             H, Ht, V, lam)
        tc = jnp.sum(sse[:, 0, 0]) / denom
        return (wn, done | (tc <= _TOL)), None

    (W_fix, _), _ = lax.scan(step, (W_pre, jnp.asarray(False)), None,
                             length=_MAXIT - 1)

    wn, _ = _iter_call(jnp.zeros((1,), jnp.int32), W_fix, L_bf, m_bf,
                       H, Ht, V, lam)
    return _K1 * wn


# speedup vs baseline: 1133.5026x; 2.0259x over previous
import jax
import jax.numpy as jnp
from jax import lax
from jax.experimental import pallas as pl
from jax.experimental.pallas import tpu as pltpu

_ROWS, _RANK, _COLS = 4096, 64, 512
_LTR = 0.01
_ALPHA = 0.1
_TOL = 1e-8
_MAXIT = 20
_K1 = 1.0
_KPOW = 32
_BLK = 512
_NBLK = _ROWS // _BLK
_CH = 1024
_NCH = _ROWS // _CH
_DENOM = float(_RANK * _ROWS)


def _setup_kernel(mt_ref, zt_ref, h_ref, hbf_ref, htbf_ref, vt_ref, lam_ref):
    h = h_ref[...]
    m32 = mt_ref[...].astype(jnp.float32)
    mbf = mt_ref[...]
    hbf = hbf_ref[...]
    htbf = htbf_ref[...]

    vt_ref[...] = jnp.dot(h, m32 * zt_ref[...],
                          preferred_element_type=jnp.float32)

    v0 = jnp.dot(h * h, m32, preferred_element_type=jnp.float32) + _ALPHA

    def body(_, v):
        t = jnp.dot(htbf, v, preferred_element_type=jnp.float32)
        tm = t.astype(jnp.bfloat16) * mbf
        w = jnp.dot(hbf, tm, preferred_element_type=jnp.float32) \
            + _ALPHA * v.astype(jnp.float32)
        return (w / jnp.max(jnp.abs(w), axis=0, keepdims=True)
                ).astype(jnp.bfloat16)

    v = lax.fori_loop(0, _KPOW, body, v0.astype(jnp.bfloat16))
    v32 = v.astype(jnp.float32)
    t = jnp.dot(htbf, v, preferred_element_type=jnp.float32)
    w = jnp.dot(h, t * m32, preferred_element_type=jnp.float32) + _ALPHA * v32
    lam_ref[...] = (jnp.sum(v32 * w, axis=0, keepdims=True)
                    / jnp.sum(v32 * v32, axis=0, keepdims=True))


def _main_kernel(lt_ref, mt_ref, hbf_ref, htbf_ref, vt_ref, lam_ref, wp_ref,
                 out_ref, w_scr, wn_scr):
    w_scr[...] = wp_ref[...]
    lam = lam_ref[...]
    vt = vt_ref[...]
    hbf = hbf_ref[...]
    htbf = htbf_ref[...]

    def step(done):
        wT = w_scr[...]
        wbf = wT.astype(jnp.bfloat16)
        sse = jnp.zeros((), jnp.float32)
        for c in range(_NCH):
            sl = slice(c * _CH, (c + 1) * _CH)
            gt = jnp.dot(wbf, lt_ref[:, sl],
                         preferred_element_type=jnp.float32)
            tt = jnp.dot(htbf, wbf[:, sl],
                         preferred_element_type=jnp.float32)
            tm = tt.astype(jnp.bfloat16) * mt_ref[:, sl]
            ut = jnp.dot(hbf, tm,
                         preferred_element_type=jnp.float32)
            wc = wT[:, sl]
            wn = jax.nn.relu(wc + (vt[:, sl] - ut - _ALPHA * wc - _LTR * gt)
                             / lam[:, sl])
            wn_scr[:, sl] = wn
            d = wn - wc
            sse = sse + jnp.sum(d * d)
        return sse

    def body(_, done):
        sse = step(done)

        @pl.when(jnp.logical_not(done))
        def _():
            w_scr[...] = wn_scr[...]

        return done | (sse / _DENOM <= _TOL)

    lax.fori_loop(0, _MAXIT - 1, body, jnp.asarray(False))
    step(jnp.asarray(False))
    out_ref[...] = wn_scr[...]


def _setup_call(mT_bf, ZT, H, H_bf, Ht_bf):
    return pl.pallas_call(
        _setup_kernel,
        grid=(_NBLK,),
        in_specs=[
            pl.BlockSpec((_COLS, _BLK), lambda i: (0, i)),
            pl.BlockSpec((_COLS, _BLK), lambda i: (0, i)),
            pl.BlockSpec((_RANK, _COLS), lambda i: (0, 0)),
            pl.BlockSpec((_RANK, _COLS), lambda i: (0, 0)),
            pl.BlockSpec((_COLS, _RANK), lambda i: (0, 0)),
        ],
        out_specs=[
            pl.BlockSpec((_RANK, _BLK), lambda i: (0, i)),
            pl.BlockSpec((1, _BLK), lambda i: (0, i)),
        ],
        out_shape=[
            jax.ShapeDtypeStruct((_RANK, _ROWS), jnp.float32),
            jax.ShapeDtypeStruct((1, _ROWS), jnp.float32),
        ],
        compiler_params=pltpu.CompilerParams(
            dimension_semantics=("parallel",),
        ),
        name="wlayer_setup",
    )(mT_bf, ZT, H, H_bf, Ht_bf)


def _main_call(LT_bf, mT_bf, H_bf, Ht_bf, VT, lamT, WpT):
    return pl.pallas_call(
        _main_kernel,
        in_specs=[
            pl.BlockSpec(memory_space=pltpu.VMEM),
            pl.BlockSpec(memory_space=pltpu.VMEM),
            pl.BlockSpec(memory_space=pltpu.VMEM),
            pl.BlockSpec(memory_space=pltpu.VMEM),
            pl.BlockSpec(memory_space=pltpu.VMEM),
            pl.BlockSpec(memory_space=pltpu.VMEM),
            pl.BlockSpec(memory_space=pltpu.VMEM),
        ],
        out_specs=pl.BlockSpec(memory_space=pltpu.VMEM),
        out_shape=jax.ShapeDtypeStruct((_RANK, _ROWS), jnp.float32),
        scratch_shapes=[
            pltpu.VMEM((_RANK, _ROWS), jnp.float32),
            pltpu.VMEM((_RANK, _ROWS), jnp.float32),
        ],
        compiler_params=pltpu.CompilerParams(
            vmem_limit_bytes=56 * 1024 * 1024,
        ),
        name="wlayer_main",
    )(LT_bf, mT_bf, H_bf, Ht_bf, VT, lamT, WpT)


def kernel(Omega, W_pre, H, L, Z):
    LT_bf = L.T.astype(jnp.bfloat16)
    mT_bf = Omega.T.astype(jnp.bfloat16)
    ZT = Z.T
    H_bf = H.astype(jnp.bfloat16)
    Ht_bf = H_bf.T
    WpT = W_pre.T

    VT, lamT = _setup_call(mT_bf, ZT, H, H_bf, Ht_bf)
    outT = _main_call(LT_bf, mT_bf, H_bf, Ht_bf, VT, lamT, WpT)
    return _K1 * outT.T


# speedup vs baseline: 1424.6856x; 1.2569x over previous
import jax
import jax.numpy as jnp
from jax import lax
from jax.experimental import pallas as pl
from jax.experimental.pallas import tpu as pltpu

_ROWS, _RANK, _COLS = 4096, 64, 512
_LTR = 0.01
_ALPHA = 0.1
_TOL = 1e-8
_MAXIT = 20
_K1 = 1.0
_KPOW = 32
_BLK = 512
_NBLK = _ROWS // _BLK
_CH = 1024
_NCH = _ROWS // _CH
_DENOM = float(_RANK * _ROWS)
_LSCALE = 256.0


def _setup_kernel(mt_ref, zt_ref, h_ref, hbf_ref, htbf_ref, vt_ref, lam_ref):
    h = h_ref[...]
    m32 = mt_ref[...].astype(jnp.float32)
    mbf = mt_ref[...]
    hbf = hbf_ref[...]
    htbf = htbf_ref[...]

    vt_ref[...] = jnp.dot(h, m32 * zt_ref[...],
                          preferred_element_type=jnp.float32)

    v0 = jnp.dot(h * h, m32, preferred_element_type=jnp.float32) + _ALPHA

    def body(_, v):
        t = jnp.dot(htbf, v, preferred_element_type=jnp.float32)
        tm = t.astype(jnp.bfloat16) * mbf
        w = jnp.dot(hbf, tm, preferred_element_type=jnp.float32) \
            + _ALPHA * v.astype(jnp.float32)
        return (w / jnp.max(jnp.abs(w), axis=0, keepdims=True)
                ).astype(jnp.bfloat16)

    v = lax.fori_loop(0, _KPOW, body, v0.astype(jnp.bfloat16))
    v32 = v.astype(jnp.float32)
    t = jnp.dot(htbf, v, preferred_element_type=jnp.float32)
    w = jnp.dot(h, t * m32, preferred_element_type=jnp.float32) + _ALPHA * v32
    lam_ref[...] = (jnp.sum(v32 * w, axis=0, keepdims=True)
                    / jnp.sum(v32 * v32, axis=0, keepdims=True))


def _main_kernel(lt_ref, mt_ref, h_ref, hbf_ref, htbf_ref, vt_ref, lam_ref,
                 wp_ref, out_ref, w_scr, wn_scr):
    w_scr[...] = wp_ref[...]
    inv_lam = 1.0 / lam_ref[...]
    vt = vt_ref[...]
    h = h_ref[...]
    hbf = hbf_ref[...]
    htbf = htbf_ref[...]
    ltr_s = _LTR / _LSCALE

    def step():
        wT = w_scr[...]
        wbf = wT.astype(jnp.bfloat16)
        wf8 = wT.astype(jnp.float8_e4m3fn)
        sse_vec = jnp.zeros((1, _CH), jnp.float32)
        for c in range(_NCH):
            sl = slice(c * _CH, (c + 1) * _CH)
            gt = jnp.dot(wf8, lt_ref[:, sl],
                         preferred_element_type=jnp.float32)
            tt = jnp.dot(htbf, wbf[:, sl],
                         preferred_element_type=jnp.float32)
            tm = jnp.where(mt_ref[:, sl] != 0, tt, 0.0)
            ut = jnp.dot(h, tm,
                         preferred_element_type=jnp.float32)
            wc = wT[:, sl]
            wn = jax.nn.relu(wc + (vt[:, sl] - ut - _ALPHA * wc - ltr_s * gt)
                             * inv_lam[:, sl])
            wn_scr[:, sl] = wn
            d = wn - wc
            sse_vec = sse_vec + jnp.sum(d * d, axis=0, keepdims=True)
        return jnp.sum(sse_vec)

    def body(_, done):
        sse = step()

        @pl.when(jnp.logical_not(done))
        def _():
            w_scr[...] = wn_scr[...]

        return done | (sse / _DENOM <= _TOL)

    lax.fori_loop(0, _MAXIT - 1, body, jnp.asarray(False))
    step()
    out_ref[...] = wn_scr[...]


def _setup_call(mT_bf, ZT, H, H_bf, Ht_bf):
    return pl.pallas_call(
        _setup_kernel,
        grid=(_NBLK,),
        in_specs=[
            pl.BlockSpec((_COLS, _BLK), lambda i: (0, i)),
            pl.BlockSpec((_COLS, _BLK), lambda i: (0, i)),
            pl.BlockSpec((_RANK, _COLS), lambda i: (0, 0)),
            pl.BlockSpec((_RANK, _COLS), lambda i: (0, 0)),
            pl.BlockSpec((_COLS, _RANK), lambda i: (0, 0)),
        ],
        out_specs=[
            pl.BlockSpec((_RANK, _BLK), lambda i: (0, i)),
            pl.BlockSpec((1, _BLK), lambda i: (0, i)),
        ],
        out_shape=[
            jax.ShapeDtypeStruct((_RANK, _ROWS), jnp.float32),
            jax.ShapeDtypeStruct((1, _ROWS), jnp.float32),
        ],
        compiler_params=pltpu.CompilerParams(
            dimension_semantics=("parallel",),
        ),
        name="wlayer_setup",
    )(mT_bf, ZT, H, H_bf, Ht_bf)


def _main_call(LT_f8, mT_bf, H, H_bf, Ht_bf, VT, lamT, WpT):
    return pl.pallas_call(
        _main_kernel,
        in_specs=[
            pl.BlockSpec(memory_space=pltpu.VMEM),
            pl.BlockSpec(memory_space=pltpu.VMEM),
            pl.BlockSpec(memory_space=pltpu.VMEM),
            pl.BlockSpec(memory_space=pltpu.VMEM),
            pl.BlockSpec(memory_space=pltpu.VMEM),
            pl.BlockSpec(memory_space=pltpu.VMEM),
            pl.BlockSpec(memory_space=pltpu.VMEM),
            pl.BlockSpec(memory_space=pltpu.VMEM),
        ],
        out_specs=pl.BlockSpec(memory_space=pltpu.VMEM),
        out_shape=jax.ShapeDtypeStruct((_RANK, _ROWS), jnp.float32),
        scratch_shapes=[
            pltpu.VMEM((_RANK, _ROWS), jnp.float32),
            pltpu.VMEM((_RANK, _ROWS), jnp.float32),
        ],
        compiler_params=pltpu.CompilerParams(
            vmem_limit_bytes=56 * 1024 * 1024,
        ),
        name="wlayer_main",
    )(LT_f8, mT_bf, H, H_bf, Ht_bf, VT, lamT, WpT)


def kernel(Omega, W_pre, H, L, Z):
    LT_f8 = (L.T * _LSCALE).astype(jnp.float8_e4m3fn)
    mT_bf = Omega.T.astype(jnp.bfloat16)
    ZT = Z.T
    H_bf = H.astype(jnp.bfloat16)
    Ht_bf = H_bf.T
    WpT = W_pre.T

    VT, lamT = _setup_call(mT_bf, ZT, H, H_bf, Ht_bf)
    outT = _main_call(LT_f8, mT_bf, H, H_bf, Ht_bf, VT, lamT, WpT)
    return _K1 * outT.T


# speedup vs baseline: 1660.7593x; 1.1657x over previous
import jax
import jax.numpy as jnp
from jax import lax
from jax.experimental import pallas as pl
from jax.experimental.pallas import tpu as pltpu

_ROWS, _RANK, _COLS = 4096, 64, 512
_LTR = 0.01
_ALPHA = 0.1
_TOL = 1e-8
_MAXIT = 20
_K1 = 1.0
_KPOW = 32
_BLK = 512
_NBLK = _ROWS // _BLK
_CH = 1024
_NCH = _ROWS // _CH
_DENOM = float(_RANK * _ROWS)
_LSCALE = 256.0


def _setup_kernel(om_ref, z_ref, h_ref, hf8_ref, htf8_ref,
                  mt_ref, vt_ref, lam_ref):
    h = h_ref[...]
    hf8 = hf8_ref[...]
    htf8 = htf8_ref[...]
    m32 = om_ref[...].astype(jnp.float32)
    mT = m32.T
    mt_ref[...] = mT.astype(jnp.bfloat16)

    vu = jnp.dot(m32 * z_ref[...], h.T,
                 preferred_element_type=jnp.float32)
    vt_ref[...] = vu.T

    v0 = jnp.dot(h * h, mT, preferred_element_type=jnp.float32) + _ALPHA

    def apply8(v8):
        t = jnp.dot(htf8, v8, preferred_element_type=jnp.float32)
        tm = jnp.where(mT != 0, t, 0.0).astype(jnp.float8_e4m3fn)
        return jnp.dot(hf8, tm, preferred_element_type=jnp.float32) \
            + _ALPHA * v8.astype(jnp.float32)

    def body(_, v8):
        w = apply8(v8)
        return (w / jnp.max(jnp.abs(w), axis=0, keepdims=True)
                ).astype(jnp.float8_e4m3fn)

    v8 = lax.fori_loop(0, _KPOW, body,
                       (v0 / jnp.max(v0, axis=0, keepdims=True)
                        ).astype(jnp.float8_e4m3fn))
    v32 = v8.astype(jnp.float32)
    t = jnp.dot(htf8, v8, preferred_element_type=jnp.float32)
    w = jnp.dot(h, t * mT, preferred_element_type=jnp.float32) + _ALPHA * v32
    lam_ref[...] = (jnp.sum(v32 * w, axis=0, keepdims=True)
                    / jnp.sum(v32 * v32, axis=0, keepdims=True))


def _main_kernel(lt_ref, mt_ref, h_ref, hbf_ref, htbf_ref, vt_ref, lam_ref,
                 wp_ref, out_ref, w_scr, wn_scr):
    w_scr[...] = wp_ref[...]
    inv_lam = 1.0 / lam_ref[...]
    vt = vt_ref[...]
    h = h_ref[...]
    hbf = hbf_ref[...]
    htbf = htbf_ref[...]
    ltr_s = _LTR / _LSCALE

    def step():
        wT = w_scr[...]
        wbf = wT.astype(jnp.bfloat16)
        wf8 = wT.astype(jnp.float8_e4m3fn)
        sse_vec = jnp.zeros((1, _CH), jnp.float32)
        for c in range(_NCH):
            sl = slice(c * _CH, (c + 1) * _CH)
            gt = jnp.dot(wf8, lt_ref[:, sl],
                         preferred_element_type=jnp.float32)
            tt = jnp.dot(htbf, wbf[:, sl],
                         preferred_element_type=jnp.float32)
            tm = jnp.where(mt_ref[:, sl] != 0, tt, 0.0)
            ut = jnp.dot(h, tm,
                         preferred_element_type=jnp.float32)
            wc = wT[:, sl]
            wn = jax.nn.relu(wc + (vt[:, sl] - ut - _ALPHA * wc - ltr_s * gt)
                             * inv_lam[:, sl])
            wn_scr[:, sl] = wn
            d = wn - wc
            sse_vec = sse_vec + jnp.sum(d * d, axis=0, keepdims=True)
        return jnp.sum(sse_vec)

    def body(_, done):
        sse = step()

        @pl.when(jnp.logical_not(done))
        def _():
            w_scr[...] = wn_scr[...]

        return done | (sse / _DENOM <= _TOL)

    lax.fori_loop(0, _MAXIT - 1, body, jnp.asarray(False))
    step()
    out_ref[...] = wn_scr[...]


def _setup_call(Omega, Z, H, H_f8, Ht_f8):
    return pl.pallas_call(
        _setup_kernel,
        grid=(_NBLK,),
        in_specs=[
            pl.BlockSpec((_BLK, _COLS), lambda i: (i, 0)),
            pl.BlockSpec((_BLK, _COLS), lambda i: (i, 0)),
            pl.BlockSpec((_RANK, _COLS), lambda i: (0, 0)),
            pl.BlockSpec((_RANK, _COLS), lambda i: (0, 0)),
            pl.BlockSpec((_COLS, _RANK), lambda i: (0, 0)),
        ],
        out_specs=[
            pl.BlockSpec((_COLS, _BLK), lambda i: (0, i)),
            pl.BlockSpec((_RANK, _BLK), lambda i: (0, i)),
            pl.BlockSpec((1, _BLK), lambda i: (0, i)),
        ],
        out_shape=[
            jax.ShapeDtypeStruct((_COLS, _ROWS), jnp.bfloat16),
            jax.ShapeDtypeStruct((_RANK, _ROWS), jnp.float32),
            jax.ShapeDtypeStruct((1, _ROWS), jnp.float32),
        ],
        compiler_params=pltpu.CompilerParams(
            dimension_semantics=("parallel",),
        ),
        name="wlayer_setup",
    )(Omega, Z, H, H_f8, Ht_f8)


def _main_call(LT_f8, mT_bf, H, H_bf, Ht_bf, VT, lamT, WpT):
    return pl.pallas_call(
        _main_kernel,
        in_specs=[
            pl.BlockSpec(memory_space=pltpu.VMEM),
            pl.BlockSpec(memory_space=pltpu.VMEM),
            pl.BlockSpec(memory_space=pltpu.VMEM),
            pl.BlockSpec(memory_space=pltpu.VMEM),
            pl.BlockSpec(memory_space=pltpu.VMEM),
            pl.BlockSpec(memory_space=pltpu.VMEM),
            pl.BlockSpec(memory_space=pltpu.VMEM),
            pl.BlockSpec(memory_space=pltpu.VMEM),
        ],
        out_specs=pl.BlockSpec(memory_space=pltpu.VMEM),
        out_shape=jax.ShapeDtypeStruct((_RANK, _ROWS), jnp.float32),
        scratch_shapes=[
            pltpu.VMEM((_RANK, _ROWS), jnp.float32),
            pltpu.VMEM((_RANK, _ROWS), jnp.float32),
        ],
        compiler_params=pltpu.CompilerParams(
            vmem_limit_bytes=56 * 1024 * 1024,
        ),
        name="wlayer_main",
    )(LT_f8, mT_bf, H, H_bf, Ht_bf, VT, lamT, WpT)


def kernel(Omega, W_pre, H, L, Z):
    LT_f8 = (L.T * _LSCALE).astype(jnp.float8_e4m3fn)
    H_bf = H.astype(jnp.bfloat16)
    Ht_bf = H_bf.T
    H_f8 = H.astype(jnp.float8_e4m3fn)
    Ht_f8 = H_f8.T
    WpT = W_pre.T

    mT_bf, VT, lamT = _setup_call(Omega, Z, H, H_f8, Ht_f8)
    outT = _main_call(LT_f8, mT_bf, H, H_bf, Ht_bf, VT, lamT, WpT)
    return _K1 * outT.T


# speedup vs baseline: 1813.0574x; 1.0917x over previous
import jax
import jax.numpy as jnp
from jax import lax
from jax.experimental import pallas as pl
from jax.experimental.pallas import tpu as pltpu

_ROWS, _RANK, _COLS = 4096, 64, 512
_LTR = 0.01
_ALPHA = 0.1
_TOL = 1e-8
_MAXIT = 20
_K1 = 1.0
_KPOW = 32
_BLK = 512
_NBLK = _ROWS // _BLK
_CH = 1024
_NCH = _ROWS // _CH
_DENOM = float(_RANK * _ROWS)
_LSCALE = 256.0


def _setup_kernel(om_ref, z_ref, h_ref, hf8_ref, htf8_ref,
                  mt_ref, vt_ref, lam_ref):
    h = h_ref[...]
    hf8 = hf8_ref[...]
    htf8 = htf8_ref[...]
    m32 = om_ref[...].astype(jnp.float32)
    mT = m32.T
    mt_ref[...] = mT.astype(jnp.bfloat16)

    vu = jnp.dot(m32 * z_ref[...], h.T,
                 preferred_element_type=jnp.float32)
    vt_ref[...] = vu.T

    v0 = jnp.dot(h * h, mT, preferred_element_type=jnp.float32) + _ALPHA

    def apply8(v8):
        t = jnp.dot(htf8, v8, preferred_element_type=jnp.float32)
        tm = (t * mT).astype(jnp.float8_e4m3fn)
        return jnp.dot(hf8, tm, preferred_element_type=jnp.float32) \
            + _ALPHA * v8.astype(jnp.float32)

    def body(_, v8):
        w = apply8(v8)
        return (w / jnp.max(jnp.abs(w), axis=0, keepdims=True)
                ).astype(jnp.float8_e4m3fn)

    v8 = lax.fori_loop(0, _KPOW, body,
                       (v0 / jnp.max(v0, axis=0, keepdims=True)
                        ).astype(jnp.float8_e4m3fn))
    v32 = v8.astype(jnp.float32)
    t = jnp.dot(htf8, v8, preferred_element_type=jnp.float32)
    w = jnp.dot(h, t * mT, preferred_element_type=jnp.float32) + _ALPHA * v32
    lam_ref[...] = (jnp.sum(v32 * w, axis=0, keepdims=True)
                    / jnp.sum(v32 * v32, axis=0, keepdims=True))


def _main_kernel(lt_ref, mt_ref, h_ref, hbf_ref, htbf_ref, vt_ref, lam_ref,
                 wp_ref, out_ref, w_scr, wn_scr):
    w_scr[...] = wp_ref[...]
    inv_lam = 1.0 / lam_ref[...]
    vt = vt_ref[...]
    h = h_ref[...]
    hbf = hbf_ref[...]
    htbf = htbf_ref[...]
    ltr_s = _LTR / _LSCALE

    def step():
        wT = w_scr[...]
        wbf = wT.astype(jnp.bfloat16)
        wf8 = wT.astype(jnp.float8_e4m3fn)
        sse_vec = jnp.zeros((1, _CH), jnp.float32)
        for c in range(_NCH):
            sl = slice(c * _CH, (c + 1) * _CH)
            gt = jnp.dot(wf8, lt_ref[:, sl],
                         preferred_element_type=jnp.float32)
            tt = jnp.dot(htbf, wbf[:, sl],
                         preferred_element_type=jnp.float32)
            tm = tt.astype(jnp.bfloat16) * mt_ref[:, sl]
            ut = jnp.dot(hbf, tm,
                         preferred_element_type=jnp.float32)
            wc = wT[:, sl]
            wn = jax.nn.relu(wc + (vt[:, sl] - ut - _ALPHA * wc - ltr_s * gt)
                             * inv_lam[:, sl])
            wn_scr[:, sl] = wn
            d = wn - wc
            sse_vec = sse_vec + jnp.sum(d * d, axis=0, keepdims=True)
        return jnp.sum(sse_vec)

    def body(_, done):
        sse = step()

        @pl.when(jnp.logical_not(done))
        def _():
            w_scr[...] = wn_scr[...]

        return done | (sse / _DENOM <= _TOL)

    lax.fori_loop(0, _MAXIT - 1, body, jnp.asarray(False))
    step()
    out_ref[...] = wn_scr[...]


def _setup_call(Omega, Z, H, H_f8, Ht_f8):
    return pl.pallas_call(
        _setup_kernel,
        grid=(_NBLK,),
        in_specs=[
            pl.BlockSpec((_BLK, _COLS), lambda i: (i, 0)),
            pl.BlockSpec((_BLK, _COLS), lambda i: (i, 0)),
            pl.BlockSpec((_RANK, _COLS), lambda i: (0, 0)),
            pl.BlockSpec((_RANK, _COLS), lambda i: (0, 0)),
            pl.BlockSpec((_COLS, _RANK), lambda i: (0, 0)),
        ],
        out_specs=[
            pl.BlockSpec((_COLS, _BLK), lambda i: (0, i)),
            pl.BlockSpec((_RANK, _BLK), lambda i: (0, i)),
            pl.BlockSpec((1, _BLK), lambda i: (0, i)),
        ],
        out_shape=[
            jax.ShapeDtypeStruct((_COLS, _ROWS), jnp.bfloat16),
            jax.ShapeDtypeStruct((_RANK, _ROWS), jnp.float32),
            jax.ShapeDtypeStruct((1, _ROWS), jnp.float32),
        ],
        compiler_params=pltpu.CompilerParams(
            dimension_semantics=("parallel",),
        ),
        name="wlayer_setup",
    )(Omega, Z, H, H_f8, Ht_f8)


def _main_call(LT_f8, mT_bf, H, H_bf, Ht_bf, VT, lamT, WpT):
    return pl.pallas_call(
        _main_kernel,
        in_specs=[
            pl.BlockSpec(memory_space=pltpu.VMEM),
            pl.BlockSpec(memory_space=pltpu.VMEM),
            pl.BlockSpec(memory_space=pltpu.VMEM),
            pl.BlockSpec(memory_space=pltpu.VMEM),
            pl.BlockSpec(memory_space=pltpu.VMEM),
            pl.BlockSpec(memory_space=pltpu.VMEM),
            pl.BlockSpec(memory_space=pltpu.VMEM),
            pl.BlockSpec(memory_space=pltpu.VMEM),
        ],
        out_specs=pl.BlockSpec(memory_space=pltpu.VMEM),
        out_shape=jax.ShapeDtypeStruct((_RANK, _ROWS), jnp.float32),
        scratch_shapes=[
            pltpu.VMEM((_RANK, _ROWS), jnp.float32),
            pltpu.VMEM((_RANK, _ROWS), jnp.float32),
        ],
        compiler_params=pltpu.CompilerParams(
            vmem_limit_bytes=56 * 1024 * 1024,
        ),
        name="wlayer_main",
    )(LT_f8, mT_bf, H, H_bf, Ht_bf, VT, lamT, WpT)


def kernel(Omega, W_pre, H, L, Z):
    LT_f8 = (L.T * _LSCALE).astype(jnp.float8_e4m3fn)
    H_bf = H.astype(jnp.bfloat16)
    Ht_bf = H_bf.T
    H_f8 = H.astype(jnp.float8_e4m3fn)
    Ht_f8 = H_f8.T
    WpT = W_pre.T

    mT_bf, VT, lamT = _setup_call(Omega, Z, H, H_f8, Ht_f8)
    outT = _main_call(LT_f8, mT_bf, H, H_bf, Ht_bf, VT, lamT, WpT)
    return _K1 * outT.T


# speedup vs baseline: 2261.2755x; 1.2472x over previous
import jax
import jax.numpy as jnp
from jax import lax
from jax.experimental import pallas as pl
from jax.experimental.pallas import tpu as pltpu

_ROWS, _RANK, _COLS = 4096, 64, 512
_LTR = 0.01
_ALPHA = 0.1
_TOL = 1e-8
_MAXIT = 20
_K1 = 1.0
_KPOW = 32
_BLK = 512
_NBLK = _ROWS // _BLK
_CH = 1024
_NCH = _ROWS // _CH
_DENOM = float(_RANK * _ROWS)
_LSCALE = 256.0


def _setup_kernel(om_ref, z_ref, h_ref, hf8_ref, htf8_ref,
                  mt_ref, vt_ref, lam_ref):
    h = h_ref[...]
    hf8 = hf8_ref[...]
    htf8 = htf8_ref[...]
    m32 = om_ref[...].astype(jnp.float32)
    mT = m32.T
    mt_ref[...] = mT.astype(jnp.bfloat16)

    vu = jnp.dot(m32 * z_ref[...], h.T,
                 preferred_element_type=jnp.float32)
    vt_ref[...] = vu.T

    v0 = jnp.dot(h * h, mT, preferred_element_type=jnp.float32) + _ALPHA

    def apply8(v8):
        t = jnp.dot(htf8, v8, preferred_element_type=jnp.float32)
        tm = (t * mT).astype(jnp.float8_e4m3fn)
        return jnp.dot(hf8, tm, preferred_element_type=jnp.float32) \
            + _ALPHA * v8.astype(jnp.float32)

    def body(_, v8):
        w = apply8(v8)
        return (w / jnp.max(jnp.abs(w), axis=0, keepdims=True)
                ).astype(jnp.float8_e4m3fn)

    v8 = lax.fori_loop(0, _KPOW, body,
                       (v0 / jnp.max(v0, axis=0, keepdims=True)
                        ).astype(jnp.float8_e4m3fn))
    v32 = v8.astype(jnp.float32)
    t = jnp.dot(htf8, v8, preferred_element_type=jnp.float32)
    w = jnp.dot(h, t * mT, preferred_element_type=jnp.float32) + _ALPHA * v32
    lam_ref[...] = (jnp.sum(v32 * w, axis=0, keepdims=True)
                    / jnp.sum(v32 * v32, axis=0, keepdims=True))


def _main_kernel(lt_ref, mt_ref, h_ref, hbf_ref, htbf_ref, vt_ref, lam_ref,
                 wp_ref, out_ref, w_scr, wn_scr):
    w_scr[...] = wp_ref[...]
    inv_lam = 1.0 / lam_ref[...]
    vt = vt_ref[...]
    h = h_ref[...]
    hbf = hbf_ref[...]
    htbf = htbf_ref[...]
    ltr_s = _LTR / _LSCALE

    def step():
        wT = w_scr[...]
        wbf = wT.astype(jnp.bfloat16)
        wf8 = wT.astype(jnp.float8_e4m3fn)
        sse_vec = jnp.zeros((1, _CH), jnp.float32)
        for c in range(_NCH):
            sl = slice(c * _CH, (c + 1) * _CH)
            gt = jnp.dot(wf8, lt_ref[:, sl],
                         preferred_element_type=jnp.float32)
            tt = jnp.dot(htbf, wbf[:, sl],
                         preferred_element_type=jnp.float32)
            tm = tt.astype(jnp.bfloat16) * mt_ref[:, sl]
            ut = jnp.dot(hbf, tm,
                         preferred_element_type=jnp.float32)
            wc = wT[:, sl]
            wn = jax.nn.relu(wc + (vt[:, sl] - ut - _ALPHA * wc - ltr_s * gt)
                             * inv_lam[:, sl])
            wn_scr[:, sl] = wn
            d = wn - wc
            sse_vec = sse_vec + jnp.sum(d * d, axis=0, keepdims=True)
        return jnp.sum(sse_vec)

    def body(_, done):
        sse = step()

        @pl.when(jnp.logical_not(done))
        def _():
            w_scr[...] = wn_scr[...]

        return done | (sse / _DENOM <= _TOL)

    lax.fori_loop(0, _MAXIT - 1, body, jnp.asarray(False))
    step()
    out_ref[...] = wn_scr[...]


def _setup_call(Omega, Z, H, H_f8, Ht_f8):
    return pl.pallas_call(
        _setup_kernel,
        in_specs=[pl.BlockSpec(memory_space=pltpu.VMEM)] * 5,
        out_specs=[pl.BlockSpec(memory_space=pltpu.VMEM)] * 3,
        out_shape=[
            jax.ShapeDtypeStruct((_COLS, _ROWS), jnp.bfloat16),
            jax.ShapeDtypeStruct((_RANK, _ROWS), jnp.float32),
            jax.ShapeDtypeStruct((1, _ROWS), jnp.float32),
        ],
        compiler_params=pltpu.CompilerParams(
            vmem_limit_bytes=56 * 1024 * 1024,
        ),
        name="wlayer_setup",
    )(Omega, Z, H, H_f8, Ht_f8)


def _main_call(LT_f8, mT_bf, H, H_bf, Ht_bf, VT, lamT, WpT):
    return pl.pallas_call(
        _main_kernel,
        in_specs=[
            pl.BlockSpec(memory_space=pltpu.VMEM),
            pl.BlockSpec(memory_space=pltpu.VMEM),
            pl.BlockSpec(memory_space=pltpu.VMEM),
            pl.BlockSpec(memory_space=pltpu.VMEM),
            pl.BlockSpec(memory_space=pltpu.VMEM),
            pl.BlockSpec(memory_space=pltpu.VMEM),
            pl.BlockSpec(memory_space=pltpu.VMEM),
            pl.BlockSpec(memory_space=pltpu.VMEM),
        ],
        out_specs=pl.BlockSpec(memory_space=pltpu.VMEM),
        out_shape=jax.ShapeDtypeStruct((_RANK, _ROWS), jnp.float32),
        scratch_shapes=[
            pltpu.VMEM((_RANK, _ROWS), jnp.float32),
            pltpu.VMEM((_RANK, _ROWS), jnp.float32),
        ],
        compiler_params=pltpu.CompilerParams(
            vmem_limit_bytes=56 * 1024 * 1024,
        ),
        name="wlayer_main",
    )(LT_f8, mT_bf, H, H_bf, Ht_bf, VT, lamT, WpT)


def kernel(Omega, W_pre, H, L, Z):
    LT_f8 = (L.T * _LSCALE).astype(jnp.float8_e4m3fn)
    H_bf = H.astype(jnp.bfloat16)
    Ht_bf = H_bf.T
    H_f8 = H.astype(jnp.float8_e4m3fn)
    Ht_f8 = H_f8.T
    WpT = W_pre.T

    mT_bf, VT, lamT = _setup_call(Omega, Z, H, H_f8, Ht_f8)
    outT = _main_call(LT_f8, mT_bf, H, H_bf, Ht_bf, VT, lamT, WpT)
    return _K1 * outT.T
